# baseline (speedup 1.0000x reference)
import jax
import jax.numpy as jnp
from jax.experimental import pallas as pl
from jax.experimental.pallas import tpu as pltpu

_ROWS = 8192
_COLS = 1024
_GRID = 16
_BLK = _ROWS // _GRID


def _tc_body(p_ref, t_ref, o_ref):
    x = p_ref[...]
    t = t_ref[...]
    ts = t * 0.125 + 0.5
    sp = jnp.log1p(jnp.exp(-jnp.abs(x)))
    loss = jnp.maximum(x, 0.0) - x * ts + sp
    loss = jnp.where(t != 0.0, loss, 0.0)
    part = jnp.sum(loss)

    @pl.when(pl.program_id(0) == 0)
    def _init():
        o_ref[0] = 0.0

    o_ref[0] += part


def kernel(pred_map, target_map):
    p = pred_map.reshape(_ROWS, _COLS)
    t = target_map.reshape(_ROWS, _COLS)
    out = pl.pallas_call(
        _tc_body,
        grid=(_GRID,),
        in_specs=[
            pl.BlockSpec((_BLK, _COLS), lambda i: (i, 0)),
            pl.BlockSpec((_BLK, _COLS), lambda i: (i, 0)),
        ],
        out_specs=pl.BlockSpec(memory_space=pltpu.SMEM),
        out_shape=jax.ShapeDtypeStruct((1,), jnp.float32),
    )(p, t)
    return out[0]


# speedup vs baseline: 2.7188x; 2.7188x over previous
import jax
import jax.numpy as jnp
from jax.experimental import pallas as pl
from jax.experimental.pallas import tpu as pltpu

_ROWS = 16384
_COLS = 512
_GRID = 32
_BLK = _ROWS // _GRID


def _tc_body(p_ref, t_ref, o_ref):
    x = p_ref[...]
    t = t_ref[...]
    ts = t * 0.125 + 0.5
    u = jnp.exp(-jnp.abs(x))
    sp = jnp.log(1.0 + u)
    loss = jnp.maximum(x, 0.0) - x * ts + sp
    loss = jnp.where(t != 0.0, loss, 0.0)
    part = jnp.sum(loss)

    @pl.when(pl.program_id(0) == 0)
    def _init():
        o_ref[0] = 0.0

    o_ref[0] += part


def kernel(pred_map, target_map):
    p = pred_map.reshape(_ROWS, _COLS)
    t = target_map.reshape(_ROWS, _COLS)
    out = pl.pallas_call(
        _tc_body,
        grid=(_GRID,),
        in_specs=[
            pl.BlockSpec((_BLK, _COLS), lambda i: (i, 0)),
            pl.BlockSpec((_BLK, _COLS), lambda i: (i, 0)),
        ],
        out_specs=pl.BlockSpec(memory_space=pltpu.SMEM),
        out_shape=jax.ShapeDtypeStruct((1,), jnp.float32),
    )(p, t)
    return out[0]
